# baseline (speedup 1.0000x reference)
import functools

import jax
import jax.numpy as jnp
from jax import lax
from jax.experimental import pallas as pl
from jax.experimental.pallas import tpu as pltpu
from jax.experimental.pallas import tpu_sc as plsc

BATCH = 4
N_INSTANCE = 1
LENGTH = 2048
HIDDEN = 1024
ROW_TILE = 2048

NUM_SC = 2
NUM_SUBCORES = 16
NW = NUM_SC * NUM_SUBCORES
ROWS = BATCH * LENGTH
RPW = ROWS // NW
CH = 32
NCH = RPW // CH
LANES = 16
NCOL = HIDDEN // LANES


def _lane_sum(v):
    dnums = lax.GatherDimensionNumbers(
        offset_dims=(), collapsed_slice_dims=(0,), start_index_map=(0,))
    for sh in (8, 4, 2, 1):
        idx = lax.iota(jnp.int32, LANES) ^ sh
        perm = lax.gather(v, idx.reshape(LANES, 1), dnums, slice_sizes=(1,),
                          mode=lax.GatherScatterMode.PROMISE_IN_BOUNDS)
        v = v + perm
    return v


def _sc_body(in_hbm, emb_hbm, s_hbm, b_hbm, out_hbm, xbuf, ebuf, sbuf, bbuf):
    wid = lax.axis_index("s") * NUM_SC + lax.axis_index("c")
    base = wid * RPW
    pltpu.sync_copy(s_hbm, sbuf)
    pltpu.sync_copy(b_hbm, bbuf)

    def chunk_body(ci, _):
        row0 = base + ci * CH
        erow0 = lax.rem(row0, LENGTH)
        pltpu.sync_copy(in_hbm.at[pl.ds(row0, CH)], xbuf)
        pltpu.sync_copy(emb_hbm.at[pl.ds(erow0, CH)], ebuf)

        def row_body(r, _):
            def col_sum(i, carry):
                s, q = carry
                x = xbuf[r, pl.ds(i * LANES, LANES)] + ebuf[r, pl.ds(i * LANES, LANES)]
                xbuf[r, pl.ds(i * LANES, LANES)] = x
                return (s + x, q + x * x)

            z = jnp.zeros((LANES,), jnp.float32)
            s, q = lax.fori_loop(0, NCOL, col_sum, (z, z))
            mv = _lane_sum(s) * (1.0 / HIDDEN)
            msq = _lane_sum(q) * (1.0 / HIDDEN)
            tv = (msq - mv * mv) + 1e-6
            iv = lax.bitcast_convert_type(tv, jnp.int32)
            iv = jnp.int32(0x5F3759DF) - (iv >> 1)
            y = lax.bitcast_convert_type(iv, jnp.float32)
            for _ in range(4):
                y = y * (1.5 - 0.5 * tv * y * y)

            def col_out(i, _):
                sv = sbuf[pl.ds(i * LANES, LANES)]
                bv = bbuf[pl.ds(i * LANES, LANES)]
                x = xbuf[r, pl.ds(i * LANES, LANES)]
                rs = y * sv
                xbuf[r, pl.ds(i * LANES, LANES)] = x * rs + (bv - mv * rs)
                return 0

            lax.fori_loop(0, NCOL, col_out, 0)
            return 0

        lax.fori_loop(0, CH, row_body, 0)
        pltpu.sync_copy(xbuf, out_hbm.at[pl.ds(row0, CH)])
        return 0

    lax.fori_loop(0, NCH, chunk_body, 0)


def _sc_call(inputs2d, embedding, ln_scale, ln_bias):
    mesh = plsc.VectorSubcoreMesh(
        core_axis_name="c", subcore_axis_name="s",
        num_cores=NUM_SC, num_subcores=NUM_SUBCORES)
    run = pl.kernel(
        _sc_body,
        out_type=jax.ShapeDtypeStruct((ROWS, HIDDEN), jnp.float32),
        mesh=mesh,
        scratch_types=[
            pltpu.VMEM((CH, HIDDEN), jnp.float32),
            pltpu.VMEM((CH, HIDDEN), jnp.float32),
            pltpu.VMEM((HIDDEN,), jnp.float32),
            pltpu.VMEM((HIDDEN,), jnp.float32),
        ],
    )
    return run(inputs2d, embedding, ln_scale, ln_bias)


def _add_body(x_ref, e_ref, s_ref, b_ref, o_ref):
    o_ref[0, 0] = x_ref[0, 0] + e_ref[...]


def _ln_body(x_ref, e_ref, s_ref, b_ref, o_ref):
    x = x_ref[0, 0] + e_ref[...]
    inv_n = 1.0 / HIDDEN
    mean = jnp.sum(x, axis=-1, keepdims=True) * inv_n
    msq = jnp.sum(x * x, axis=-1, keepdims=True) * inv_n
    var = msq - mean * mean
    r = jax.lax.rsqrt(var + 1e-6)
    scale = r * s_ref[0]
    shift = b_ref[0] - (r * mean) * s_ref[0]
    o_ref[0, 0] = x * scale + shift


_BODY = _ln_body

TILE2 = 2048


def _ln_body2(x_ref, e_ref, s_ref, b_ref, o_ref):
    l0 = lax.rem(pl.program_id(0), LENGTH // TILE2) * TILE2
    x = x_ref[...] + e_ref[pl.ds(l0, TILE2), :]
    inv_n = 1.0 / HIDDEN
    mean = jnp.sum(x, axis=-1, keepdims=True) * inv_n
    msq = jnp.sum(x * x, axis=-1, keepdims=True) * inv_n
    var = msq - mean * mean
    r = jax.lax.rsqrt(var + 1e-6)
    scale = r * s_ref[0]
    shift = b_ref[0] - (r * mean) * s_ref[0]
    o_ref[...] = x * scale + shift


CCH = 128


def _ln_body3(x_ref, e_ref, s_ref, b_ref, o_ref):
    l0 = lax.rem(pl.program_id(0), LENGTH // TILE2) * TILE2
    acc_s = jnp.zeros((TILE2, CCH), jnp.float32)
    acc_q = jnp.zeros((TILE2, CCH), jnp.float32)
    for c in range(HIDDEN // CCH):
        sl = pl.ds(c * CCH, CCH)
        xc = x_ref[:, sl] + e_ref[pl.ds(l0, TILE2), sl]
        o_ref[:, sl] = xc
        acc_s = acc_s + xc
        acc_q = acc_q + xc * xc
    inv_n = 1.0 / HIDDEN
    mean = jnp.sum(acc_s, axis=-1, keepdims=True) * inv_n
    msq = jnp.sum(acc_q, axis=-1, keepdims=True) * inv_n
    var = msq - mean * mean
    r = jax.lax.rsqrt(var + 1e-6)
    for c in range(HIDDEN // CCH):
        sl = pl.ds(c * CCH, CCH)
        scale = r * s_ref[0, sl]
        shift = b_ref[0, sl] - (r * mean) * s_ref[0, sl]
        o_ref[:, sl] = o_ref[:, sl] * scale + shift


def _tc_call2(inputs, embedding, ln_scale, ln_bias):
    x2 = inputs.reshape(ROWS, HIDDEN)
    out = pl.pallas_call(
        _ln_body3,
        grid=(ROWS // TILE2,),
        in_specs=[
            pl.BlockSpec((TILE2, HIDDEN), lambda i: (i, 0)),
            pl.BlockSpec((LENGTH, HIDDEN), lambda i: (0, 0)),
            pl.BlockSpec((1, HIDDEN), lambda i: (0, 0)),
            pl.BlockSpec((1, HIDDEN), lambda i: (0, 0)),
        ],
        out_specs=pl.BlockSpec((TILE2, HIDDEN), lambda i: (i, 0)),
        out_shape=jax.ShapeDtypeStruct((ROWS, HIDDEN), jnp.float32),
    )(x2, embedding, ln_scale.reshape(1, HIDDEN), ln_bias.reshape(1, HIDDEN))
    return out.reshape(BATCH, N_INSTANCE, LENGTH, HIDDEN)


MCH = 1024
NBUF = 5
NCHUNKS = ROWS // MCH


def _man_ln(xsrc, esrc, sb, bb, odst):
    acc_s = jnp.zeros((MCH, CCH), jnp.float32)
    acc_q = jnp.zeros((MCH, CCH), jnp.float32)
    for c in range(HIDDEN // CCH):
        sl = pl.ds(c * CCH, CCH)
        xc = xsrc[:, sl] + esrc[:, sl]
        odst[:, sl] = xc
        acc_s = acc_s + xc
        acc_q = acc_q + xc * xc
    inv_n = 1.0 / HIDDEN
    mean = jnp.sum(acc_s, axis=-1, keepdims=True) * inv_n
    msq = jnp.sum(acc_q, axis=-1, keepdims=True) * inv_n
    r = jax.lax.rsqrt((msq - mean * mean) + 1e-6)
    for c in range(HIDDEN // CCH):
        sl = pl.ds(c * CCH, CCH)
        scale = r * sb[0, sl]
        shift = bb[0, sl] - (r * mean) * sb[0, sl]
        odst[:, sl] = odst[:, sl] * scale + shift


def _man_body(x_hbm, e_hbm, s_hbm, b_hbm, o_hbm,
              xb, eb, sb, bb, ob, insem, outsem, esem, ssem, bsem):
    pltpu.make_async_copy(e_hbm, eb, esem).start()
    pltpu.make_async_copy(s_hbm, sb, ssem).start()
    pltpu.make_async_copy(b_hbm, bb, bsem).start()
    for k in range(NBUF):
        pltpu.make_async_copy(
            x_hbm.at[pl.ds(k * MCH, MCH)], xb.at[k], insem.at[k]).start()
    pltpu.make_async_copy(e_hbm, eb, esem).wait()
    pltpu.make_async_copy(s_hbm, sb, ssem).wait()
    pltpu.make_async_copy(b_hbm, bb, bsem).wait()
    for ci in range(NCHUNKS):
        slot = ci % NBUF
        pltpu.make_async_copy(
            x_hbm.at[pl.ds(ci * MCH, MCH)], xb.at[slot], insem.at[slot]).wait()
        if ci >= NBUF:
            prev = ci - NBUF
            pltpu.make_async_copy(
                ob.at[slot], o_hbm.at[pl.ds(prev * MCH, MCH)], outsem.at[slot]).wait()
        l0 = (ci * MCH) % LENGTH
        _man_ln(xb.at[slot], eb.at[pl.ds(l0, MCH)], sb, bb, ob.at[slot])
        pltpu.make_async_copy(
            ob.at[slot], o_hbm.at[pl.ds(ci * MCH, MCH)], outsem.at[slot]).start()
        nxt = ci + NBUF
        if nxt < NCHUNKS:
            pltpu.make_async_copy(
                x_hbm.at[pl.ds(nxt * MCH, MCH)], xb.at[slot], insem.at[slot]).start()
    for ci in range(NCHUNKS - NBUF, NCHUNKS):
        slot = ci % NBUF
        pltpu.make_async_copy(
            ob.at[slot], o_hbm.at[pl.ds(ci * MCH, MCH)], outsem.at[slot]).wait()


def _tc_call3(inputs, embedding, ln_scale, ln_bias):
    x2 = inputs.reshape(ROWS, HIDDEN)
    out = pl.pallas_call(
        _man_body,
        in_specs=[
            pl.BlockSpec(memory_space=pl.MemorySpace.ANY),
            pl.BlockSpec(memory_space=pl.MemorySpace.ANY),
            pl.BlockSpec(memory_space=pl.MemorySpace.ANY),
            pl.BlockSpec(memory_space=pl.MemorySpace.ANY),
        ],
        out_specs=pl.BlockSpec(memory_space=pl.MemorySpace.ANY),
        out_shape=jax.ShapeDtypeStruct((ROWS, HIDDEN), jnp.float32),
        scratch_shapes=[
            pltpu.VMEM((NBUF, MCH, HIDDEN), jnp.float32),
            pltpu.VMEM((LENGTH, HIDDEN), jnp.float32),
            pltpu.VMEM((1, HIDDEN), jnp.float32),
            pltpu.VMEM((1, HIDDEN), jnp.float32),
            pltpu.VMEM((NBUF, MCH, HIDDEN), jnp.float32),
            pltpu.SemaphoreType.DMA((NBUF,)),
            pltpu.SemaphoreType.DMA((NBUF,)),
            pltpu.SemaphoreType.DMA,
            pltpu.SemaphoreType.DMA,
            pltpu.SemaphoreType.DMA,
        ],
    )(x2, embedding, ln_scale.reshape(1, HIDDEN), ln_bias.reshape(1, HIDDEN))
    return out.reshape(BATCH, N_INSTANCE, LENGTH, HIDDEN)


def _tc_call(inputs, embedding, ln_scale, ln_bias):
    grid = (LENGTH // ROW_TILE, BATCH)
    return pl.pallas_call(
        _BODY,
        grid=grid,
        in_specs=[
            pl.BlockSpec((1, 1, ROW_TILE, HIDDEN), lambda l, b: (b, 0, l, 0)),
            pl.BlockSpec((ROW_TILE, HIDDEN), lambda l, b: (l, 0)),
            pl.BlockSpec((1, HIDDEN), lambda l, b: (0, 0)),
            pl.BlockSpec((1, HIDDEN), lambda l, b: (0, 0)),
        ],
        out_specs=pl.BlockSpec((1, 1, ROW_TILE, HIDDEN), lambda l, b: (b, 0, l, 0)),
        out_shape=jax.ShapeDtypeStruct((BATCH, N_INSTANCE, LENGTH, HIDDEN), jnp.float32),
    )(inputs, embedding, ln_scale.reshape(1, HIDDEN), ln_bias.reshape(1, HIDDEN))


def kernel(inputs, embedding, ln_scale, ln_bias):
    return _tc_call3(inputs, embedding, ln_scale, ln_bias)


# speedup vs baseline: 1.0373x; 1.0373x over previous
import jax
import jax.numpy as jnp
from jax import lax
from jax.experimental import pallas as pl
from jax.experimental.pallas import tpu as pltpu

BATCH = 4
N_INSTANCE = 1
LENGTH = 2048
HIDDEN = 1024
ROWS = BATCH * LENGTH

CCH = 128
MCH = 512
NBUF = 8
NCHUNKS = ROWS // MCH


def _chunk_layernorm(xsrc, esrc, sb, bb, odst):
    acc_s = jnp.zeros((MCH, CCH), jnp.float32)
    acc_q = jnp.zeros((MCH, CCH), jnp.float32)
    for c in range(HIDDEN // CCH):
        sl = pl.ds(c * CCH, CCH)
        xc = xsrc[:, sl] + esrc[:, sl]
        odst[:, sl] = xc
        acc_s = acc_s + xc
        acc_q = acc_q + xc * xc
    inv_n = 1.0 / HIDDEN
    mean = jnp.sum(acc_s, axis=-1, keepdims=True) * inv_n
    msq = jnp.sum(acc_q, axis=-1, keepdims=True) * inv_n
    r = jax.lax.rsqrt((msq - mean * mean) + 1e-6)
    for c in range(HIDDEN // CCH):
        sl = pl.ds(c * CCH, CCH)
        scale = r * sb[0, sl]
        shift = bb[0, sl] - (r * mean) * sb[0, sl]
        odst[:, sl] = odst[:, sl] * scale + shift


def _body(x_hbm, e_hbm, s_hbm, b_hbm, o_hbm,
          xb, eb, sb, bb, ob, insem, outsem, esem, ssem, bsem):
    pltpu.make_async_copy(e_hbm, eb, esem).start()
    pltpu.make_async_copy(s_hbm, sb, ssem).start()
    pltpu.make_async_copy(b_hbm, bb, bsem).start()
    for k in range(NBUF):
        pltpu.make_async_copy(
            x_hbm.at[pl.ds(k * MCH, MCH)], xb.at[k], insem.at[k]).start()
    pltpu.make_async_copy(e_hbm, eb, esem).wait()
    pltpu.make_async_copy(s_hbm, sb, ssem).wait()
    pltpu.make_async_copy(b_hbm, bb, bsem).wait()
    for ci in range(NCHUNKS):
        slot = ci % NBUF
        pltpu.make_async_copy(
            x_hbm.at[pl.ds(ci * MCH, MCH)], xb.at[slot], insem.at[slot]).wait()
        if ci >= NBUF:
            prev = ci - NBUF
            pltpu.make_async_copy(
                ob.at[slot], o_hbm.at[pl.ds(prev * MCH, MCH)], outsem.at[slot]).wait()
        l0 = (ci * MCH) % LENGTH
        _chunk_layernorm(xb.at[slot], eb.at[pl.ds(l0, MCH)], sb, bb, ob.at[slot])
        pltpu.make_async_copy(
            ob.at[slot], o_hbm.at[pl.ds(ci * MCH, MCH)], outsem.at[slot]).start()
        nxt = ci + NBUF
        if nxt < NCHUNKS:
            pltpu.make_async_copy(
                x_hbm.at[pl.ds(nxt * MCH, MCH)], xb.at[slot], insem.at[slot]).start()
    for ci in range(NCHUNKS - NBUF, NCHUNKS):
        slot = ci % NBUF
        pltpu.make_async_copy(
            ob.at[slot], o_hbm.at[pl.ds(ci * MCH, MCH)], outsem.at[slot]).wait()


def kernel(inputs, embedding, ln_scale, ln_bias):
    x2 = inputs.reshape(ROWS, HIDDEN)
    out = pl.pallas_call(
        _body,
        in_specs=[
            pl.BlockSpec(memory_space=pl.MemorySpace.ANY),
            pl.BlockSpec(memory_space=pl.MemorySpace.ANY),
            pl.BlockSpec(memory_space=pl.MemorySpace.ANY),
            pl.BlockSpec(memory_space=pl.MemorySpace.ANY),
        ],
        out_specs=pl.BlockSpec(memory_space=pl.MemorySpace.ANY),
        out_shape=jax.ShapeDtypeStruct((ROWS, HIDDEN), jnp.float32),
        scratch_shapes=[
            pltpu.VMEM((NBUF, MCH, HIDDEN), jnp.float32),
            pltpu.VMEM((LENGTH, HIDDEN), jnp.float32),
            pltpu.VMEM((1, HIDDEN), jnp.float32),
            pltpu.VMEM((1, HIDDEN), jnp.float32),
            pltpu.VMEM((NBUF, MCH, HIDDEN), jnp.float32),
            pltpu.SemaphoreType.DMA((NBUF,)),
            pltpu.SemaphoreType.DMA((NBUF,)),
            pltpu.SemaphoreType.DMA,
            pltpu.SemaphoreType.DMA,
            pltpu.SemaphoreType.DMA,
        ],
    )(x2, embedding, ln_scale.reshape(1, HIDDEN), ln_bias.reshape(1, HIDDEN))
    return out.reshape(BATCH, N_INSTANCE, LENGTH, HIDDEN)


# speedup vs baseline: 1.0417x; 1.0042x over previous
import jax
import jax.numpy as jnp
from jax import lax
from jax.experimental import pallas as pl
from jax.experimental.pallas import tpu as pltpu

BATCH = 4
N_INSTANCE = 1
LENGTH = 2048
HIDDEN = 1024
ROWS = BATCH * LENGTH

CCH = 128
MCH = 512
NBUF = 8
NCHUNKS = ROWS // MCH
NEPART = LENGTH // MCH


def _chunk_layernorm(xsrc, esrc, sb, bb, odst):
    acc_s = jnp.zeros((MCH, CCH), jnp.float32)
    acc_q = jnp.zeros((MCH, CCH), jnp.float32)
    for c in range(HIDDEN // CCH):
        sl = pl.ds(c * CCH, CCH)
        xc = xsrc[:, sl] + esrc[:, sl]
        odst[:, sl] = xc
        acc_s = acc_s + xc
        acc_q = acc_q + xc * xc
    inv_n = 1.0 / HIDDEN
    mean = jnp.sum(acc_s, axis=-1, keepdims=True) * inv_n
    msq = jnp.sum(acc_q, axis=-1, keepdims=True) * inv_n
    r = jax.lax.rsqrt((msq - mean * mean) + 1e-6)
    for c in range(HIDDEN // CCH):
        sl = pl.ds(c * CCH, CCH)
        scale = r * sb[0, sl]
        shift = bb[0, sl] - (r * mean) * sb[0, sl]
        odst[:, sl] = odst[:, sl] * scale + shift


def _body(x_hbm, e_hbm, s_hbm, b_hbm, o_hbm,
          xb, eb, sb, bb, ob, insem, outsem, esem, ssem, bsem):
    for k in range(NEPART):
        pltpu.make_async_copy(
            e_hbm.at[pl.ds(k * MCH, MCH)], eb.at[pl.ds(k * MCH, MCH)],
            esem.at[k]).start()
    pltpu.make_async_copy(s_hbm, sb, ssem).start()
    pltpu.make_async_copy(b_hbm, bb, bsem).start()
    for k in range(NBUF):
        pltpu.make_async_copy(
            x_hbm.at[pl.ds(k * MCH, MCH)], xb.at[k], insem.at[k]).start()
    pltpu.make_async_copy(s_hbm, sb, ssem).wait()
    pltpu.make_async_copy(b_hbm, bb, bsem).wait()
    for ci in range(NCHUNKS):
        slot = ci % NBUF
        if ci < NEPART:
            pltpu.make_async_copy(
                e_hbm.at[pl.ds(ci * MCH, MCH)], eb.at[pl.ds(ci * MCH, MCH)],
                esem.at[ci]).wait()
        pltpu.make_async_copy(
            x_hbm.at[pl.ds(ci * MCH, MCH)], xb.at[slot], insem.at[slot]).wait()
        if ci >= NBUF:
            prev = ci - NBUF
            pltpu.make_async_copy(
                ob.at[slot], o_hbm.at[pl.ds(prev * MCH, MCH)], outsem.at[slot]).wait()
        l0 = (ci * MCH) % LENGTH
        _chunk_layernorm(xb.at[slot], eb.at[pl.ds(l0, MCH)], sb, bb, ob.at[slot])
        pltpu.make_async_copy(
            ob.at[slot], o_hbm.at[pl.ds(ci * MCH, MCH)], outsem.at[slot]).start()
        nxt = ci + NBUF
        if nxt < NCHUNKS:
            pltpu.make_async_copy(
                x_hbm.at[pl.ds(nxt * MCH, MCH)], xb.at[slot], insem.at[slot]).start()
    for ci in range(NCHUNKS - NBUF, NCHUNKS):
        slot = ci % NBUF
        pltpu.make_async_copy(
            ob.at[slot], o_hbm.at[pl.ds(ci * MCH, MCH)], outsem.at[slot]).wait()


def kernel(inputs, embedding, ln_scale, ln_bias):
    x2 = inputs.reshape(ROWS, HIDDEN)
    out = pl.pallas_call(
        _body,
        in_specs=[
            pl.BlockSpec(memory_space=pl.MemorySpace.ANY),
            pl.BlockSpec(memory_space=pl.MemorySpace.ANY),
            pl.BlockSpec(memory_space=pl.MemorySpace.ANY),
            pl.BlockSpec(memory_space=pl.MemorySpace.ANY),
        ],
        out_specs=pl.BlockSpec(memory_space=pl.MemorySpace.ANY),
        out_shape=jax.ShapeDtypeStruct((ROWS, HIDDEN), jnp.float32),
        scratch_shapes=[
            pltpu.VMEM((NBUF, MCH, HIDDEN), jnp.float32),
            pltpu.VMEM((LENGTH, HIDDEN), jnp.float32),
            pltpu.VMEM((1, HIDDEN), jnp.float32),
            pltpu.VMEM((1, HIDDEN), jnp.float32),
            pltpu.VMEM((NBUF, MCH, HIDDEN), jnp.float32),
            pltpu.SemaphoreType.DMA((NBUF,)),
            pltpu.SemaphoreType.DMA((NBUF,)),
            pltpu.SemaphoreType.DMA((NEPART,)),
            pltpu.SemaphoreType.DMA,
            pltpu.SemaphoreType.DMA,
        ],
    )(x2, embedding, ln_scale.reshape(1, HIDDEN), ln_bias.reshape(1, HIDDEN))
    return out.reshape(BATCH, N_INSTANCE, LENGTH, HIDDEN)


# speedup vs baseline: 1.0809x; 1.0377x over previous
import jax
import jax.numpy as jnp
from jax import lax
from jax.experimental import pallas as pl
from jax.experimental.pallas import tpu as pltpu

BATCH = 4
N_INSTANCE = 1
LENGTH = 2048
HIDDEN = 1024
ROWS = BATCH * LENGTH

CCH = 128
MCH = 512
NBUF = 8
NCHUNKS = ROWS // MCH
NEPART = LENGTH // MCH


def _chunk_layernorm(xsrc, esrc, sb, bb, odst):
    acc_s = jnp.zeros((MCH, CCH), jnp.float32)
    acc_q = jnp.zeros((MCH, CCH), jnp.float32)
    for c in range(HIDDEN // CCH):
        sl = pl.ds(c * CCH, CCH)
        xc = xsrc[:, sl] + esrc[:, sl]
        odst[:, sl] = xc
        acc_s = acc_s + xc
        acc_q = acc_q + xc * xc
    inv_n = 1.0 / HIDDEN
    mean = jnp.sum(acc_s, axis=-1, keepdims=True) * inv_n
    msq = jnp.sum(acc_q, axis=-1, keepdims=True) * inv_n
    r = jax.lax.rsqrt((msq - mean * mean) + 1e-6)
    for c in range(HIDDEN // CCH):
        sl = pl.ds(c * CCH, CCH)
        scale = r * sb[0, sl]
        shift = bb[0, sl] - (r * mean) * sb[0, sl]
        odst[:, sl] = odst[:, sl] * scale + shift


def _body(x_hbm, e_hbm, s_hbm, b_hbm, o_hbm,
          xb, eb, sb, bb, ob, insem, outsem, esem, ssem, bsem):
    pltpu.make_async_copy(s_hbm, sb, ssem).start()
    pltpu.make_async_copy(b_hbm, bb, bsem).start()
    for k in range(NBUF):
        if k < NEPART:
            pltpu.make_async_copy(
                e_hbm.at[pl.ds(k * MCH, MCH)], eb.at[pl.ds(k * MCH, MCH)],
                esem.at[k]).start()
        pltpu.make_async_copy(
            x_hbm.at[pl.ds(k * MCH, MCH)], xb.at[k], insem.at[k]).start()
    pltpu.make_async_copy(s_hbm, sb, ssem).wait()
    pltpu.make_async_copy(b_hbm, bb, bsem).wait()
    for ci in range(NCHUNKS):
        slot = ci % NBUF
        if ci < NEPART:
            pltpu.make_async_copy(
                e_hbm.at[pl.ds(ci * MCH, MCH)], eb.at[pl.ds(ci * MCH, MCH)],
                esem.at[ci]).wait()
        pltpu.make_async_copy(
            x_hbm.at[pl.ds(ci * MCH, MCH)], xb.at[slot], insem.at[slot]).wait()
        if ci >= NBUF:
            prev = ci - NBUF
            pltpu.make_async_copy(
                ob.at[slot], o_hbm.at[pl.ds(prev * MCH, MCH)], outsem.at[slot]).wait()
        l0 = (ci * MCH) % LENGTH
        _chunk_layernorm(xb.at[slot], eb.at[pl.ds(l0, MCH)], sb, bb, ob.at[slot])
        pltpu.make_async_copy(
            ob.at[slot], o_hbm.at[pl.ds(ci * MCH, MCH)], outsem.at[slot]).start()
        nxt = ci + NBUF
        if nxt < NCHUNKS:
            pltpu.make_async_copy(
                x_hbm.at[pl.ds(nxt * MCH, MCH)], xb.at[slot], insem.at[slot]).start()
    for ci in range(NCHUNKS - NBUF, NCHUNKS):
        slot = ci % NBUF
        pltpu.make_async_copy(
            ob.at[slot], o_hbm.at[pl.ds(ci * MCH, MCH)], outsem.at[slot]).wait()


def kernel(inputs, embedding, ln_scale, ln_bias):
    x2 = inputs.reshape(ROWS, HIDDEN)
    out = pl.pallas_call(
        _body,
        in_specs=[
            pl.BlockSpec(memory_space=pl.MemorySpace.ANY),
            pl.BlockSpec(memory_space=pl.MemorySpace.ANY),
            pl.BlockSpec(memory_space=pl.MemorySpace.ANY),
            pl.BlockSpec(memory_space=pl.MemorySpace.ANY),
        ],
        out_specs=pl.BlockSpec(memory_space=pl.MemorySpace.ANY),
        out_shape=jax.ShapeDtypeStruct((ROWS, HIDDEN), jnp.float32),
        scratch_shapes=[
            pltpu.VMEM((NBUF, MCH, HIDDEN), jnp.float32),
            pltpu.VMEM((LENGTH, HIDDEN), jnp.float32),
            pltpu.VMEM((1, HIDDEN), jnp.float32),
            pltpu.VMEM((1, HIDDEN), jnp.float32),
            pltpu.VMEM((NBUF, MCH, HIDDEN), jnp.float32),
            pltpu.SemaphoreType.DMA((NBUF,)),
            pltpu.SemaphoreType.DMA((NBUF,)),
            pltpu.SemaphoreType.DMA((NEPART,)),
            pltpu.SemaphoreType.DMA,
            pltpu.SemaphoreType.DMA,
        ],
    )(x2, embedding, ln_scale.reshape(1, HIDDEN), ln_bias.reshape(1, HIDDEN))
    return out.reshape(BATCH, N_INSTANCE, LENGTH, HIDDEN)
